# Initial kernel scaffold; baseline (speedup 1.0000x reference)
import jax
import jax.numpy as jnp
from jax.experimental import pallas as pl


def kernel(patch_logits, coords, output_size):
    raise NotImplementedError("write your pallas kernel here")


# speedup vs baseline: 14.7589x; 14.7589x over previous
import functools

import jax
import jax.numpy as jnp
from jax import lax
from jax.experimental import pallas as pl
from jax.experimental.pallas import tpu as pltpu
from jax.experimental.pallas import tpu_sc as plsc

_MIN_COV = 1e-06
_FILL = -10.0


def _prefix16(m, iota):
    s = jnp.where(m, 1, 0).astype(jnp.int32)
    for sh in (1, 2, 4, 8):
        idx = jnp.maximum(iota - sh, 0)
        g = s.at[idx].get(mode="promise_in_bounds")
        s = s + jnp.where(iota >= sh, g, 0)
    return s


def _sc_scatter(p2, rs, cs, B, K, C, ps, H, W):
    NB = H // ps
    TASKS = B * NB
    NW = 32
    TPW = TASKS // NW
    BAND_W = C * ps * W
    CNT_W = ps * W
    CH = 16

    mesh = plsc.VectorSubcoreMesh(core_axis_name="c", subcore_axis_name="s")

    @functools.partial(
        pl.kernel,
        out_type=(
            jax.ShapeDtypeStruct((TASKS * BAND_W,), jnp.float32),
            jax.ShapeDtypeStruct((TASKS * CNT_W,), jnp.float32),
        ),
        mesh=mesh,
        compiler_params=pltpu.CompilerParams(needs_layout_passes=False),
        scratch_types=[
            pltpu.VMEM((BAND_W,), jnp.float32),
            pltpu.VMEM((CNT_W,), jnp.float32),
            pltpu.VMEM((CH, C * ps * ps), jnp.float32),
            pltpu.VMEM((K + 16,), jnp.int32),
            pltpu.VMEM((K + 16,), jnp.int32),
            pltpu.VMEM((K + 16,), jnp.int32),
            pltpu.VMEM((K + 16,), jnp.int32),
            pltpu.VMEM((K + 16,), jnp.int32),
            pltpu.SemaphoreType.DMA,
        ],
    )
    def k(p2_hbm, rs_hbm, cs_hbm, sums_hbm, cnts_hbm,
          acc, cac, stage, rsv, csv, ids, rsel, csel, sem):
        wid = lax.axis_index("s") * 2 + lax.axis_index("c")
        b = wid // 4
        bk = b * K
        pltpu.sync_copy(rs_hbm.at[pl.ds(bk, K)], rsv.at[pl.ds(0, K)])
        pltpu.sync_copy(cs_hbm.at[pl.ds(bk, K)], csv.at[pl.ds(0, K)])
        zero16 = jnp.zeros((16,), jnp.float32)
        one16 = jnp.ones((16,), jnp.float32)
        iota = lax.iota(jnp.int32, 16)

        def task_body(j, _):
            band = (wid % 4) * TPW + j
            h0 = band * ps
            task = b * NB + band

            def zacc(i, carry):
                acc[pl.ds(i * 16, 16)] = zero16
                return carry

            lax.fori_loop(0, BAND_W // 16, zacc, 0)

            def zcnt(i, carry):
                cac[pl.ds(i * 16, 16)] = zero16
                return carry

            lax.fori_loop(0, CNT_W // 16, zcnt, 0)

            def zids(i, carry):
                ids[pl.ds(i * 16, 16)] = jnp.full((16,), bk, jnp.int32)
                return carry

            lax.fori_loop(0, (K + 16) // 16, zids, 0)

            def sel(kk, cnt):
                rv = rsv[pl.ds(kk * 16, 16)]
                cv = csv[pl.ds(kk * 16, 16)]
                m = (rv >= h0 - (ps - 1)) & (rv <= h0 + (ps - 1))
                pre = _prefix16(m, iota)
                idx = cnt + pre - 1
                plsc.store_scatter(ids, [idx], bk + kk * 16 + iota, mask=m)
                plsc.store_scatter(rsel, [idx], rv, mask=m)
                plsc.store_scatter(csel, [idx], cv, mask=m)
                return cnt + pre[15]

            n = lax.fori_loop(0, K // 16, sel, jnp.int32(0))
            nch = (n + CH - 1) // CH

            def chunk_body(ch, carry):
                pltpu.async_copy(p2_hbm.at[ids.at[pl.ds(ch * CH, CH)]],
                                 stage, sem).wait()

                def p_body(p, c2):
                    gp = ch * CH + p

                    @pl.when(gp < n)
                    def _():
                        r = rsel[pl.ds(gp, 16)][0]
                        c0 = csel[pl.ds(gp, 16)][0]
                        lo = jnp.maximum(r, h0)
                        hi = jnp.minimum(r + ps, h0 + ps)

                        def row_body(i, c3):
                            li = i - h0
                            pi = i - r
                            plsc.addupdate(cac.at[pl.ds(li * W + c0, 16)],
                                           one16)
                            for c in range(C):
                                v = stage[p, pl.ds((c * ps + pi) * ps, ps)]
                                plsc.addupdate(
                                    acc.at[pl.ds((c * ps + li) * W + c0, 16)],
                                    v)
                            return c3

                        lax.fori_loop(lo, hi, row_body, 0)

                    return c2

                lax.fori_loop(0, CH, p_body, 0)
                return carry

            lax.fori_loop(0, nch, chunk_body, 0)
            pltpu.sync_copy(acc, sums_hbm.at[pl.ds(task * BAND_W, BAND_W)])
            pltpu.sync_copy(cac, cnts_hbm.at[pl.ds(task * CNT_W, CNT_W)])
            return _

        lax.fori_loop(0, TPW, task_body, 0)

    return k(p2, rs, cs)


def _normalize(sums5, cnts5, B, C, H, W, ps):
    NB = H // ps
    NBB = 8

    def body(s_ref, c_ref, o_ref):
        s = s_ref[0, :, 0].reshape(NBB * ps, W)
        cnt = c_ref[0, :, 0].reshape(NBB * ps, W)
        covered = cnt > _MIN_COV
        o_ref[0, 0] = jnp.where(
            covered, s / jnp.maximum(cnt, _MIN_COV),
            jnp.full_like(s, _FILL))

    return pl.pallas_call(
        body,
        grid=(B, C, NB // NBB),
        in_specs=[
            pl.BlockSpec((1, NBB, 1, ps, W), lambda b, c, n: (b, n, c, 0, 0)),
            pl.BlockSpec((1, NBB, 1, ps, W), lambda b, c, n: (b, n, 0, 0, 0)),
        ],
        out_specs=pl.BlockSpec((1, 1, NBB * ps, W),
                               lambda b, c, n: (b, c, n, 0)),
        out_shape=jax.ShapeDtypeStruct((B, C, H, W), jnp.float32),
    )(sums5, cnts5)


def kernel(patch_logits, coords, output_size):
    B, K, C, ps, _ = patch_logits.shape
    H, W = 512, 512
    p2 = patch_logits.reshape(B * K, C * ps * ps)
    rs = coords[:, :, 0].reshape(-1)
    cs = coords[:, :, 1].reshape(-1)
    sums_flat, cnts_flat = _sc_scatter(p2, rs, cs, B, K, C, ps, H, W)
    NB = H // ps
    sums5 = sums_flat.reshape(B, NB, C, ps, W)
    cnts5 = cnts_flat.reshape(B, NB, 1, ps, W)
    return _normalize(sums5, cnts5, B, C, H, W, ps)


# speedup vs baseline: 17.3671x; 1.1767x over previous
import functools

import jax
import jax.numpy as jnp
from jax import lax
from jax.experimental import pallas as pl
from jax.experimental.pallas import tpu as pltpu
from jax.experimental.pallas import tpu_sc as plsc

_MIN_COV = 1e-06
_FILL = -10.0


def _prefix16(m, iota):
    s = jnp.where(m, 1, 0).astype(jnp.int32)
    for sh in (1, 2, 4, 8):
        idx = jnp.maximum(iota - sh, 0)
        g = s.at[idx].get(mode="promise_in_bounds")
        s = s + jnp.where(iota >= sh, g, 0)
    return s


def _sc_scatter(p2, rs, cs, B, K, C, ps, H, W):
    NB = H // ps
    TASKS = B * NB
    NW = 32
    TPW = TASKS // NW
    BAND_W = C * ps * W
    CNT_W = ps * W
    CH = 8

    mesh = plsc.VectorSubcoreMesh(core_axis_name="c", subcore_axis_name="s")

    @functools.partial(
        pl.kernel,
        out_type=(
            jax.ShapeDtypeStruct((TASKS * BAND_W,), jnp.float32),
            jax.ShapeDtypeStruct((TASKS * CNT_W,), jnp.float32),
        ),
        mesh=mesh,
        compiler_params=pltpu.CompilerParams(needs_layout_passes=False),
        scratch_types=[
            pltpu.VMEM((BAND_W,), jnp.float32),
            pltpu.VMEM((CNT_W,), jnp.float32),
            pltpu.VMEM((CH, C * ps * ps), jnp.float32),
            pltpu.VMEM((K + 16,), jnp.int32),
            pltpu.VMEM((K + 16,), jnp.int32),
            pltpu.VMEM((K + 16,), jnp.int32),
            pltpu.VMEM((K + 16,), jnp.int32),
            pltpu.VMEM((K + 16,), jnp.int32),
            pltpu.SemaphoreType.DMA,
        ],
    )
    def k(p2_hbm, rs_hbm, cs_hbm, sums_hbm, cnts_hbm,
          acc, cac, stage, rsv, csv, ids, rsel, csel, sem):
        wid = lax.axis_index("s") * 2 + lax.axis_index("c")
        b = wid // 4
        bk = b * K
        pltpu.sync_copy(rs_hbm.at[pl.ds(bk, K)], rsv.at[pl.ds(0, K)])
        pltpu.sync_copy(cs_hbm.at[pl.ds(bk, K)], csv.at[pl.ds(0, K)])
        zero16 = jnp.zeros((16,), jnp.float32)
        one16 = jnp.ones((16,), jnp.float32)
        iota = lax.iota(jnp.int32, 16)

        def zids(i, carry):
            ids[pl.ds(i * 16, 16)] = jnp.full((16,), bk, jnp.int32)
            return carry

        lax.fori_loop(0, (K + 16) // 16, zids, 0)

        def task_body(j, carry0):
            band = (wid % 4) * TPW + j
            h0 = band * ps
            task = b * NB + band

            def zacc(i, carry):
                for u in range(8):
                    acc[pl.ds(i * 128 + u * 16, 16)] = zero16
                return carry

            lax.fori_loop(0, BAND_W // 128, zacc, 0)

            def zcnt(i, carry):
                for u in range(8):
                    cac[pl.ds(i * 128 + u * 16, 16)] = zero16
                return carry

            lax.fori_loop(0, CNT_W // 128, zcnt, 0)

            def sel(kk, cnt):
                rv = rsv[pl.ds(kk * 16, 16)]
                cv = csv[pl.ds(kk * 16, 16)]
                m = (rv >= h0 - (ps - 1)) & (rv <= h0 + (ps - 1))
                pre = _prefix16(m, iota)
                idx = cnt + pre - 1
                plsc.store_scatter(ids, [idx], bk + kk * 16 + iota, mask=m)
                plsc.store_scatter(rsel, [idx], rv, mask=m)
                plsc.store_scatter(csel, [idx], cv, mask=m)
                return cnt + pre[15]

            n = lax.fori_loop(0, K // 16, sel, jnp.int32(0))
            nch = (n + CH - 1) // CH

            def chunk_body(ch, carry):
                pltpu.async_copy(p2_hbm.at[ids.at[pl.ds(ch * CH, CH)]],
                                 stage, sem).wait()

                def p_body(p, c2):
                    gp = ch * CH + p

                    @pl.when(gp < n)
                    def _():
                        r = rsel[pl.ds(gp, 16)][0]
                        c0 = csel[pl.ds(gp, 16)][0]
                        lo = jnp.maximum(r, h0)
                        hi = jnp.minimum(r + ps, h0 + ps)

                        def row_body(i, c3):
                            li = i - h0
                            pi = i - r
                            plsc.addupdate(cac.at[pl.ds(li * W + c0, 16)],
                                           one16)
                            for c in range(C):
                                v = stage[p, pl.ds((c * ps + pi) * ps, ps)]
                                plsc.addupdate(
                                    acc.at[pl.ds((c * ps + li) * W + c0, 16)],
                                    v)
                            return c3

                        lax.fori_loop(lo, hi, row_body, 0)

                    return c2

                lax.fori_loop(0, CH, p_body, 0)
                return carry

            lax.fori_loop(0, nch, chunk_body, 0)
            pltpu.sync_copy(acc, sums_hbm.at[pl.ds(task * BAND_W, BAND_W)])
            pltpu.sync_copy(cac, cnts_hbm.at[pl.ds(task * CNT_W, CNT_W)])
            return carry0

        lax.fori_loop(0, TPW, task_body, 0)

    return k(p2, rs, cs)


def _normalize(sums5, cnts5, B, C, H, W, ps):
    NB = H // ps
    NBB = 8

    def body(s_ref, c_ref, o_ref):
        s = s_ref[0, :, 0].reshape(NBB * ps, W)
        cnt = c_ref[0, :, 0].reshape(NBB * ps, W)
        covered = cnt > _MIN_COV
        o_ref[0, 0] = jnp.where(
            covered, s / jnp.maximum(cnt, _MIN_COV),
            jnp.full_like(s, _FILL))

    return pl.pallas_call(
        body,
        grid=(B, C, NB // NBB),
        in_specs=[
            pl.BlockSpec((1, NBB, 1, ps, W), lambda b, c, n: (b, n, c, 0, 0)),
            pl.BlockSpec((1, NBB, 1, ps, W), lambda b, c, n: (b, n, 0, 0, 0)),
        ],
        out_specs=pl.BlockSpec((1, 1, NBB * ps, W),
                               lambda b, c, n: (b, c, n, 0)),
        out_shape=jax.ShapeDtypeStruct((B, C, H, W), jnp.float32),
    )(sums5, cnts5)


def kernel(patch_logits, coords, output_size):
    B, K, C, ps, _ = patch_logits.shape
    H, W = 512, 512
    p2 = patch_logits.reshape(B * K, C * ps * ps)
    rs = coords[:, :, 0].reshape(-1)
    cs = coords[:, :, 1].reshape(-1)
    sums_flat, cnts_flat = _sc_scatter(p2, rs, cs, B, K, C, ps, H, W)
    NB = H // ps
    sums5 = sums_flat.reshape(B, NB, C, ps, W)
    cnts5 = cnts_flat.reshape(B, NB, 1, ps, W)
    return _normalize(sums5, cnts5, B, C, H, W, ps)


# speedup vs baseline: 19.4060x; 1.1174x over previous
import functools

import jax
import jax.numpy as jnp
from jax import lax
from jax.experimental import pallas as pl
from jax.experimental.pallas import tpu as pltpu
from jax.experimental.pallas import tpu_sc as plsc

_MIN_COV = 1e-06
_FILL = -10.0


def _prefix16(m, iota):
    s = jnp.where(m, 1, 0).astype(jnp.int32)
    for sh in (1, 2, 4, 8):
        idx = jnp.maximum(iota - sh, 0)
        g = s.at[idx].get(mode="promise_in_bounds")
        s = s + jnp.where(iota >= sh, g, 0)
    return s


def _sc_scatter(p2, rs, cs, B, K, C, ps, H, W):
    NB = H // ps
    TASKS = B * NB
    NW = 32
    TPW = TASKS // NW
    BAND_W = C * ps * W
    CNT_W = ps * W
    CH = 8

    mesh = plsc.VectorSubcoreMesh(core_axis_name="c", subcore_axis_name="s")

    @functools.partial(
        pl.kernel,
        out_type=(
            jax.ShapeDtypeStruct((TASKS * BAND_W,), jnp.float32),
            jax.ShapeDtypeStruct((TASKS * CNT_W,), jnp.float32),
        ),
        mesh=mesh,
        compiler_params=pltpu.CompilerParams(needs_layout_passes=False),
        scratch_types=[
            pltpu.VMEM((BAND_W,), jnp.float32),
            pltpu.VMEM((CNT_W,), jnp.float32),
            pltpu.VMEM((CH, C * ps * ps), jnp.float32),
            pltpu.VMEM((CH, C * ps * ps), jnp.float32),
            pltpu.VMEM((K + 16,), jnp.int32),
            pltpu.VMEM((K + 16,), jnp.int32),
            pltpu.VMEM((K + 16,), jnp.int32),
            pltpu.VMEM((K + 16,), jnp.int32),
            pltpu.VMEM((K + 16,), jnp.int32),
            pltpu.SemaphoreType.DMA,
            pltpu.SemaphoreType.DMA,
        ],
    )
    def k(p2_hbm, rs_hbm, cs_hbm, sums_hbm, cnts_hbm,
          acc, cac, stage_a, stage_b, rsv, csv, ids, rsel, csel,
          sem_a, sem_b):
        wid = lax.axis_index("s") * 2 + lax.axis_index("c")
        b = wid // 4
        bk = b * K
        pltpu.sync_copy(rs_hbm.at[pl.ds(bk, K)], rsv.at[pl.ds(0, K)])
        pltpu.sync_copy(cs_hbm.at[pl.ds(bk, K)], csv.at[pl.ds(0, K)])
        zero16 = jnp.zeros((16,), jnp.float32)
        one16 = jnp.ones((16,), jnp.float32)
        iota = lax.iota(jnp.int32, 16)

        def zids(i, carry):
            ids[pl.ds(i * 16, 16)] = jnp.full((16,), bk, jnp.int32)
            return carry

        lax.fori_loop(0, (K + 16) // 16, zids, 0)

        def task_body(j, carry0):
            band = (wid % 4) * TPW + j
            h0 = band * ps
            task = b * NB + band

            def zacc(i, carry):
                for u in range(8):
                    acc[pl.ds(i * 128 + u * 16, 16)] = zero16
                return carry

            lax.fori_loop(0, BAND_W // 128, zacc, 0)

            def zcnt(i, carry):
                for u in range(8):
                    cac[pl.ds(i * 128 + u * 16, 16)] = zero16
                return carry

            lax.fori_loop(0, CNT_W // 128, zcnt, 0)

            def sel(kk, cnt):
                rv = rsv[pl.ds(kk * 16, 16)]
                cv = csv[pl.ds(kk * 16, 16)]
                m = (rv >= h0 - (ps - 1)) & (rv <= h0 + (ps - 1))
                pre = _prefix16(m, iota)
                idx = cnt + pre - 1
                plsc.store_scatter(ids, [idx], bk + kk * 16 + iota, mask=m)
                plsc.store_scatter(rsel, [idx], rv, mask=m)
                plsc.store_scatter(csel, [idx], cv, mask=m)
                return cnt + pre[15]

            n = lax.fori_loop(0, K // 16, sel, jnp.int32(0))
            nch = (n + CH - 1) // CH

            def gather_start(ch, stage, sem):
                pltpu.async_copy(p2_hbm.at[ids.at[pl.ds(ch * CH, CH)]],
                                 stage, sem)

            def gather_wait(ch, stage, sem):
                pltpu.make_async_copy(p2_hbm.at[ids.at[pl.ds(ch * CH, CH)]],
                                      stage, sem).wait()

            def process(ch, stage):
                def p_body(p, c2):
                    gp = ch * CH + p

                    @pl.when(gp < n)
                    def _():
                        r = rsel[pl.ds(gp, 16)][0]
                        c0 = csel[pl.ds(gp, 16)][0]
                        lo = jnp.maximum(r, h0)
                        hi = jnp.minimum(r + ps, h0 + ps)

                        def row_body(i, c3):
                            li = i - h0
                            pi = i - r
                            plsc.addupdate(cac.at[pl.ds(li * W + c0, 16)],
                                           one16)
                            for c in range(C):
                                v = stage[p, pl.ds((c * ps + pi) * ps, ps)]
                                plsc.addupdate(
                                    acc.at[pl.ds((c * ps + li) * W + c0, 16)],
                                    v)
                            return c3

                        lax.fori_loop(lo, hi, row_body, 0)

                    return c2

                lax.fori_loop(0, CH, p_body, 0)

            @pl.when(nch > 0)
            def _():
                gather_start(0, stage_a, sem_a)

                def pair_body(cp, carry):
                    e = 2 * cp
                    o = e + 1

                    @pl.when(o < nch)
                    def _():
                        gather_start(o, stage_b, sem_b)

                    gather_wait(e, stage_a, sem_a)
                    process(e, stage_a)

                    @pl.when(o < nch)
                    def _():
                        @pl.when(o + 1 < nch)
                        def _():
                            gather_start(o + 1, stage_a, sem_a)

                        gather_wait(o, stage_b, sem_b)
                        process(o, stage_b)

                    return carry

                lax.fori_loop(0, (nch + 1) // 2, pair_body, 0)
            pltpu.sync_copy(acc, sums_hbm.at[pl.ds(task * BAND_W, BAND_W)])
            pltpu.sync_copy(cac, cnts_hbm.at[pl.ds(task * CNT_W, CNT_W)])
            return carry0

        lax.fori_loop(0, TPW, task_body, 0)

    return k(p2, rs, cs)


def _normalize(sums5, cnts5, B, C, H, W, ps):
    NB = H // ps
    NBB = 8

    def body(s_ref, c_ref, o_ref):
        s = s_ref[0, :, 0].reshape(NBB * ps, W)
        cnt = c_ref[0, :, 0].reshape(NBB * ps, W)
        covered = cnt > _MIN_COV
        o_ref[0, 0] = jnp.where(
            covered, s / jnp.maximum(cnt, _MIN_COV),
            jnp.full_like(s, _FILL))

    return pl.pallas_call(
        body,
        grid=(B, C, NB // NBB),
        in_specs=[
            pl.BlockSpec((1, NBB, 1, ps, W), lambda b, c, n: (b, n, c, 0, 0)),
            pl.BlockSpec((1, NBB, 1, ps, W), lambda b, c, n: (b, n, 0, 0, 0)),
        ],
        out_specs=pl.BlockSpec((1, 1, NBB * ps, W),
                               lambda b, c, n: (b, c, n, 0)),
        out_shape=jax.ShapeDtypeStruct((B, C, H, W), jnp.float32),
    )(sums5, cnts5)


def kernel(patch_logits, coords, output_size):
    B, K, C, ps, _ = patch_logits.shape
    H, W = 512, 512
    p2 = patch_logits.reshape(B * K, C * ps * ps)
    rs = coords[:, :, 0].reshape(-1)
    cs = coords[:, :, 1].reshape(-1)
    sums_flat, cnts_flat = _sc_scatter(p2, rs, cs, B, K, C, ps, H, W)
    NB = H // ps
    sums5 = sums_flat.reshape(B, NB, C, ps, W)
    cnts5 = cnts_flat.reshape(B, NB, 1, ps, W)
    return _normalize(sums5, cnts5, B, C, H, W, ps)
